# baseline (speedup 1.0000x reference)
import jax
import jax.numpy as jnp
from jax.experimental import pallas as pl
from jax.experimental.pallas import tpu as pltpu

_BLOCK_ROWS = 29128


def _norm_body(x_ref, o_ref):
    x = x_ref[...]
    s = jnp.sum(x, axis=1, keepdims=True)
    o_ref[...] = x * (1.0 / s)


def kernel(x):
    n, b = x.shape
    grid = (pl.cdiv(n, _BLOCK_ROWS),)
    return pl.pallas_call(
        _norm_body,
        grid=grid,
        in_specs=[pl.BlockSpec((_BLOCK_ROWS, b), lambda i: (i, 0))],
        out_specs=pl.BlockSpec((_BLOCK_ROWS, b), lambda i: (i, 0)),
        out_shape=jax.ShapeDtypeStruct(x.shape, x.dtype),
        compiler_params=pltpu.CompilerParams(
            dimension_semantics=("parallel",),
        ),
    )(x)
